# baseline (speedup 1.0000x reference)
import functools

import jax
import jax.numpy as jnp
from jax import lax
from jax.experimental import pallas as pl
from jax.experimental.pallas import tpu as pltpu
from jax.experimental.pallas import tpu_sc as plsc

_LANES = 16


def _make_gather(B, D):
    info = plsc.get_sparse_core_info()
    NC, NS = info.num_cores, info.num_subcores
    NW = NC * NS
    assert B % (NW * _LANES) == 0
    b_per_w = B // NW
    mesh = plsc.VectorSubcoreMesh(core_axis_name="c", subcore_axis_name="s")

    @functools.partial(
        pl.kernel,
        out_type=jax.ShapeDtypeStruct((B, D), jnp.float32),
        mesh=mesh,
        scratch_types=[
            pltpu.VMEM((b_per_w,), jnp.int32),
            pltpu.VMEM((b_per_w, D), jnp.float32),
            pltpu.SemaphoreType.DMA,
            pltpu.SemaphoreType.DMA,
        ],
    )
    def gather_kernel(idx_hbm, table_hbm, out_hbm, idx_v, rows_v, sem,
                      idx_sem):
        wid = lax.axis_index("s") * NC + lax.axis_index("c")
        base = wid * b_per_w
        n_ichunk = 4
        ichunk = b_per_w // n_ichunk
        idx_copies = [
            pltpu.async_copy(
                idx_hbm.at[pl.ds(base + k * ichunk, ichunk)],
                idx_v.at[pl.ds(k * ichunk, ichunk)],
                idx_sem,
            )
            for k in range(n_ichunk)
        ]

        def block_body(i, carry):
            idx16 = idx_v[pl.ds(i * _LANES, _LANES)]
            for j in range(_LANES):
                pltpu.async_copy(
                    table_hbm.at[pl.ds(idx16[j], 1)],
                    rows_v.at[pl.ds(i * _LANES + j, 1)],
                    sem,
                )
            return carry

        blocks_per_ichunk = ichunk // _LANES
        for k in range(n_ichunk):
            idx_copies[k].wait()
            lax.fori_loop(k * blocks_per_ichunk, (k + 1) * blocks_per_ichunk,
                          block_body, 0, unroll=2)

        pltpu.make_async_copy(
            table_hbm.at[pl.ds(0, b_per_w)], rows_v, sem
        ).wait()

        pltpu.sync_copy(rows_v, out_hbm.at[pl.ds(base, b_per_w)])

    return gather_kernel


def kernel(task_ids, table):
    (B,) = task_ids.shape
    V, D = table.shape
    return _make_gather(B, D)(task_ids.astype(jnp.int32), table)


# speedup vs baseline: 1.0092x; 1.0092x over previous
import functools

import jax
import jax.numpy as jnp
from jax import lax
from jax.experimental import pallas as pl
from jax.experimental.pallas import tpu as pltpu
from jax.experimental.pallas import tpu_sc as plsc

_LANES = 16


def _make_gather(B, D):
    info = plsc.get_sparse_core_info()
    NC, NS = info.num_cores, info.num_subcores
    NW = NC * NS
    assert B % (NW * _LANES) == 0
    b_per_w = B // NW
    mesh = plsc.VectorSubcoreMesh(core_axis_name="c", subcore_axis_name="s")

    @functools.partial(
        pl.kernel,
        out_type=jax.ShapeDtypeStruct((B, D), jnp.float32),
        mesh=mesh,
        scratch_types=[
            pltpu.VMEM((b_per_w,), jnp.int32),
            pltpu.VMEM((b_per_w, D), jnp.float32),
            pltpu.SemaphoreType.DMA,
        ],
    )
    def gather_kernel(idx_hbm, table_hbm, out_hbm, idx_v, rows_v, sem):
        wid = lax.axis_index("s") * NC + lax.axis_index("c")
        base = wid * b_per_w
        pltpu.sync_copy(idx_hbm.at[pl.ds(base, b_per_w)], idx_v)

        def block_body(i, carry):
            idx16 = idx_v[pl.ds(i * _LANES, _LANES)]
            for j in range(_LANES):
                pltpu.async_copy(
                    table_hbm.at[pl.ds(idx16[j], 1)],
                    rows_v.at[pl.ds(i * _LANES + j, 1)],
                    sem,
                )
            return carry

        lax.fori_loop(0, b_per_w // _LANES, block_body, 0, unroll=2)

        pltpu.make_async_copy(
            table_hbm.at[pl.ds(0, b_per_w)], rows_v, sem
        ).wait()

        pltpu.sync_copy(rows_v, out_hbm.at[pl.ds(base, b_per_w)])

    return gather_kernel


def kernel(task_ids, table):
    (B,) = task_ids.shape
    V, D = table.shape
    return _make_gather(B, D)(task_ids.astype(jnp.int32), table)
